# baseline (speedup 1.0000x reference)
import functools

import jax
import jax.numpy as jnp
from jax import lax
from jax.experimental import pallas as pl
from jax.experimental.pallas import tpu as pltpu
from jax.experimental.pallas import tpu_sc as plsc

ITEM_NUM = 1000000
EMBED = 96
FEATURE_GROUPS = [20, 20, 10, 10, 2, 2, 2, 1, 1, 1]
F = sum(FEATURE_GROUPS)
G = len(FEATURE_GROUPS)
B = 1024

NC, NS = 2, 16
NW = NC * NS
ROWS_W = (B * F) // NW
CHUNK = 96
NCHUNK = ROWS_W // CHUNK
LROWS = B // NW

BN_S = 0.9999950000374997


def _gather_sc(table, idx_user, idx_label):
    mesh = plsc.VectorSubcoreMesh(core_axis_name="c", subcore_axis_name="s")

    @functools.partial(
        pl.kernel,
        mesh=mesh,
        out_type=[
            jax.ShapeDtypeStruct((B * F, 128), jnp.float32),
            jax.ShapeDtypeStruct((B, 128), jnp.float32),
        ],
        scratch_types=[
            pltpu.VMEM((NCHUNK, CHUNK), jnp.int32),
            pltpu.VMEM((CHUNK, 128), jnp.float32),
            pltpu.VMEM((CHUNK, 128), jnp.float32),
            pltpu.VMEM((LROWS,), jnp.int32),
            pltpu.VMEM((LROWS, 128), jnp.float32),
            pltpu.SemaphoreType.DMA,
            pltpu.SemaphoreType.DMA,
        ],
    )
    def k(table_hbm, idxu_hbm, idxl_hbm, out_u, out_l,
          idx_v, buf0, buf1, idxl_v, lbuf, gsem, wsem):
        wid = lax.axis_index("s") * NC + lax.axis_index("c")
        base = wid * ROWS_W

        pltpu.sync_copy(idxl_hbm.at[pl.ds(wid * LROWS, LROWS)], idxl_v)
        pltpu.async_copy(table_hbm.at[idxl_v], lbuf, gsem).wait()
        pltpu.sync_copy(lbuf, out_l.at[pl.ds(wid * LROWS, LROWS)])

        pltpu.sync_copy(idxu_hbm.at[wid], idx_v)
        bufs = (buf0, buf1)
        pending = [None, None]
        for c in range(NCHUNK):
            b = bufs[c % 2]
            if pending[c % 2] is not None:
                pending[c % 2].wait()
            pltpu.async_copy(table_hbm.at[idx_v.at[c]], b, gsem).wait()
            wb = pltpu.async_copy(
                b, out_u.at[pl.ds(base + c * CHUNK, CHUNK)], wsem)
            pending[c % 2] = wb
        pending[0].wait()
        pending[1].wait()

    return k(table, idx_user, idx_label)


def _dice(x, alpha):
    xp = 1.0 / (1.0 + jnp.exp(-x))
    return alpha * (1.0 - xp) * x + xp * x


def _tc_body(gid_ref, ue_ref, le_ref, W1_ref, b1_ref, alpha1_ref,
             W2_ref, b2_ref, W3_ref, b3_ref,
             Wf1_ref, bf1_ref, af1_ref, Wf2_ref, bf2_ref, af2_ref,
             Wf3_ref, bf3_ref, out_ref, acc_ref, aq_ref):
    f = pl.program_id(0)
    le = le_ref[...][:, :EMBED]
    u = ue_ref[0][:, :EMBED]
    W1 = W1_ref[...]

    @pl.when(f == 0)
    def _init():
        acc_ref[...] = jnp.zeros_like(acc_ref)
        Wq = W1[0:EMBED] + W1[2 * EMBED:3 * EMBED]
        aq_ref[...] = jnp.dot(le, Wq, preferred_element_type=jnp.float32)

    Wu = W1[EMBED:2 * EMBED] - W1[2 * EMBED:3 * EMBED]
    Wm = W1[3 * EMBED:4 * EMBED]
    h1 = (aq_ref[...]
          + jnp.dot(u, Wu, preferred_element_type=jnp.float32)
          + jnp.dot(le * u, Wm, preferred_element_type=jnp.float32)
          + b1_ref[...])
    h1 = _dice(h1, alpha1_ref[...])
    W23 = jnp.dot(W2_ref[...], W3_ref[...],
                  preferred_element_type=jnp.float32)
    c23 = jnp.dot(b2_ref[...], W3_ref[...],
                  preferred_element_type=jnp.float32) + b3_ref[...]
    att = jnp.dot(h1, W23, preferred_element_type=jnp.float32) + c23
    pre = u * att

    g = gid_ref[f]
    acc_ref[g] = acc_ref[g] + pre

    @pl.when(f == F - 1)
    def _final():
        Wf1 = Wf1_ref[...]
        h = jnp.dot(le, Wf1[G * EMBED:], preferred_element_type=jnp.float32)
        for g2 in range(G):
            h = h + jnp.dot(acc_ref[g2], Wf1[g2 * EMBED:(g2 + 1) * EMBED],
                            preferred_element_type=jnp.float32)
        h = (h + bf1_ref[...]) * BN_S
        h = _dice(h, af1_ref[...])
        h = (jnp.dot(h, Wf2_ref[...], preferred_element_type=jnp.float32)
             + bf2_ref[...]) * BN_S
        h = _dice(h, af2_ref[...])
        out_ref[...] = (jnp.dot(h, Wf3_ref[...],
                                preferred_element_type=jnp.float32)
                        + bf3_ref[...])


def _tc_forward(ue, le, gids, W1, b1, alpha1, W2, b2, W3, b3,
                Wf1, bf1, af1, Wf2, bf2, af2, Wf3, bf3):
    def full(shape):
        return pl.BlockSpec(shape, lambda f, gid: (0,) * len(shape))
    grid_spec = pltpu.PrefetchScalarGridSpec(
        num_scalar_prefetch=1,
        grid=(F,),
        in_specs=[
            pl.BlockSpec((1, B, 128), lambda f, gid: (f, 0, 0)),
            full((B, 128)),
            full((4 * EMBED, 64)),
            full((1, 64)),
            full((1, 64)),
            full((64, 16)),
            full((1, 16)),
            full((16, 1)),
            full((1, 1)),
            full(((G + 1) * EMBED, 200)),
            full((1, 200)),
            full((1, 200)),
            full((200, 80)),
            full((1, 80)),
            full((1, 80)),
            full((80, 1)),
            full((1, 1)),
        ],
        out_specs=pl.BlockSpec((B, 1), lambda f, gid: (0, 0)),
        scratch_shapes=[
            pltpu.VMEM((G, B, EMBED), jnp.float32),
            pltpu.VMEM((B, 64), jnp.float32),
        ],
    )
    return pl.pallas_call(
        _tc_body,
        grid_spec=grid_spec,
        out_shape=jax.ShapeDtypeStruct((B, 1), jnp.float32),
    )(gids, ue, le, W1, b1.reshape(1, -1), alpha1.reshape(1, -1),
      W2, b2.reshape(1, -1), W3, b3.reshape(1, -1),
      Wf1, bf1.reshape(1, -1), af1.reshape(1, -1),
      Wf2, bf2.reshape(1, -1), af2.reshape(1, -1),
      Wf3, bf3.reshape(1, -1))


def kernel(batch_user, batch_label, table, W1, b1, alpha1, W2, b2, W3, b3,
           Wf1, bf1, af1, Wf2, bf2, af2, Wf3, bf3):
    idx_user = batch_user.astype(jnp.int32).T.reshape(NW, NCHUNK, CHUNK)
    idx_label = batch_label.astype(jnp.int32).reshape(-1)

    tpad = jnp.pad(table, ((0, 0), (0, 32)))
    ue_flat, le = _gather_sc(tpad, idx_user, idx_label)
    ue = ue_flat.reshape(F, B, 128)

    gid_list = []
    for g, n in enumerate(FEATURE_GROUPS):
        gid_list += [g] * n
    gids = jnp.asarray(gid_list, dtype=jnp.int32)

    return _tc_forward(ue, le, gids, W1, b1, alpha1, W2, b2, W3, b3,
                       Wf1, bf1, af1, Wf2, bf2, af2, Wf3, bf3)


# speedup vs baseline: 3.0760x; 3.0760x over previous
import functools

import jax
import jax.numpy as jnp
from jax import lax
from jax.experimental import pallas as pl
from jax.experimental.pallas import tpu as pltpu
from jax.experimental.pallas import tpu_sc as plsc

ITEM_NUM = 1000000
EMBED = 96
FEATURE_GROUPS = [20, 20, 10, 10, 2, 2, 2, 1, 1, 1]
F = sum(FEATURE_GROUPS)
G = len(FEATURE_GROUPS)
B = 1024

NC, NS = 2, 16
NW = NC * NS
ROWS_W = (B * F) // NW
CHUNK = 96
NCHUNK = ROWS_W // CHUNK
LROWS = B // NW

BN_S = 0.9999950000374997


TBL = 2048
NTB = -(-(ITEM_NUM + 1) // TBL)
TROWS = NTB * TBL


def _transpose_pad_tc(tableT, eye):
    def body(tT_ref, eye_ref, out_ref):
        x = tT_ref[...]
        xT = jax.lax.dot_general(
            x, eye_ref[...], (((0,), (0,)), ((), ())),
            preferred_element_type=jnp.float32)
        out_ref[:, :EMBED] = xT
        out_ref[:, EMBED:] = jnp.zeros((TBL, 128 - EMBED), jnp.float32)

    return pl.pallas_call(
        body,
        grid=(NTB,),
        in_specs=[pl.BlockSpec((EMBED, TBL), lambda t: (0, t)),
                  pl.BlockSpec((EMBED, EMBED), lambda t: (0, 0))],
        out_specs=pl.BlockSpec((TBL, 128), lambda t: (t, 0)),
        out_shape=jax.ShapeDtypeStruct((TROWS, 128), jnp.float32),
    )(tableT, eye)


def _gather_sc(table, idx_user, idx_label):
    mesh = plsc.VectorSubcoreMesh(core_axis_name="c", subcore_axis_name="s")

    @functools.partial(
        pl.kernel,
        mesh=mesh,
        out_type=[
            jax.ShapeDtypeStruct((B * F, 128), jnp.float32),
            jax.ShapeDtypeStruct((B, 128), jnp.float32),
        ],
        scratch_types=[
            pltpu.VMEM((NCHUNK, CHUNK), jnp.int32),
            pltpu.VMEM((CHUNK, 128), jnp.float32),
            pltpu.VMEM((CHUNK, 128), jnp.float32),
            pltpu.VMEM((LROWS,), jnp.int32),
            pltpu.VMEM((LROWS, 128), jnp.float32),
            pltpu.SemaphoreType.DMA,
            pltpu.SemaphoreType.DMA,
        ],
    )
    def k(table_hbm, idxu_hbm, idxl_hbm, out_u, out_l,
          idx_v, buf0, buf1, idxl_v, lbuf, gsem, wsem):
        wid = lax.axis_index("s") * NC + lax.axis_index("c")
        base = wid * ROWS_W

        pltpu.sync_copy(idxl_hbm.at[pl.ds(wid * LROWS, LROWS)], idxl_v)
        pltpu.async_copy(table_hbm.at[idxl_v], lbuf, gsem).wait()
        pltpu.sync_copy(lbuf, out_l.at[pl.ds(wid * LROWS, LROWS)])

        pltpu.sync_copy(idxu_hbm.at[wid], idx_v)
        bufs = (buf0, buf1)
        pending = [None, None]
        for c in range(NCHUNK):
            b = bufs[c % 2]
            if pending[c % 2] is not None:
                pending[c % 2].wait()
            pltpu.async_copy(table_hbm.at[idx_v.at[c]], b, gsem).wait()
            wb = pltpu.async_copy(
                b, out_u.at[pl.ds(base + c * CHUNK, CHUNK)], wsem)
            pending[c % 2] = wb
        pending[0].wait()
        pending[1].wait()

    return k(table, idx_user, idx_label)


def _dice(x, alpha):
    xp = 1.0 / (1.0 + jnp.exp(-x))
    return alpha * (1.0 - xp) * x + xp * x


def _tc_body(gid_ref, ue_ref, le_ref, W1_ref, b1_ref, alpha1_ref,
             W2_ref, b2_ref, W3_ref, b3_ref,
             Wf1_ref, bf1_ref, af1_ref, Wf2_ref, bf2_ref, af2_ref,
             Wf3_ref, bf3_ref, out_ref, acc_ref, aq_ref):
    f = pl.program_id(0)
    le = le_ref[...][:, :EMBED]
    u = ue_ref[0][:, :EMBED]
    W1 = W1_ref[...]

    @pl.when(f == 0)
    def _init():
        acc_ref[...] = jnp.zeros_like(acc_ref)
        Wq = W1[0:EMBED] + W1[2 * EMBED:3 * EMBED]
        aq_ref[...] = jnp.dot(le, Wq, preferred_element_type=jnp.float32)

    Wu = W1[EMBED:2 * EMBED] - W1[2 * EMBED:3 * EMBED]
    Wm = W1[3 * EMBED:4 * EMBED]
    h1 = (aq_ref[...]
          + jnp.dot(u, Wu, preferred_element_type=jnp.float32)
          + jnp.dot(le * u, Wm, preferred_element_type=jnp.float32)
          + b1_ref[...])
    h1 = _dice(h1, alpha1_ref[...])
    W23 = jnp.dot(W2_ref[...], W3_ref[...],
                  preferred_element_type=jnp.float32)
    c23 = jnp.dot(b2_ref[...], W3_ref[...],
                  preferred_element_type=jnp.float32) + b3_ref[...]
    att = jnp.dot(h1, W23, preferred_element_type=jnp.float32) + c23
    pre = u * att

    g = gid_ref[f]
    acc_ref[g] = acc_ref[g] + pre

    @pl.when(f == F - 1)
    def _final():
        Wf1 = Wf1_ref[...]
        h = jnp.dot(le, Wf1[G * EMBED:], preferred_element_type=jnp.float32)
        for g2 in range(G):
            h = h + jnp.dot(acc_ref[g2], Wf1[g2 * EMBED:(g2 + 1) * EMBED],
                            preferred_element_type=jnp.float32)
        h = (h + bf1_ref[...]) * BN_S
        h = _dice(h, af1_ref[...])
        h = (jnp.dot(h, Wf2_ref[...], preferred_element_type=jnp.float32)
             + bf2_ref[...]) * BN_S
        h = _dice(h, af2_ref[...])
        out_ref[...] = (jnp.dot(h, Wf3_ref[...],
                                preferred_element_type=jnp.float32)
                        + bf3_ref[...])


def _tc_forward(ue, le, gids, W1, b1, alpha1, W2, b2, W3, b3,
                Wf1, bf1, af1, Wf2, bf2, af2, Wf3, bf3):
    def full(shape):
        return pl.BlockSpec(shape, lambda f, gid: (0,) * len(shape))
    grid_spec = pltpu.PrefetchScalarGridSpec(
        num_scalar_prefetch=1,
        grid=(F,),
        in_specs=[
            pl.BlockSpec((1, B, 128), lambda f, gid: (f, 0, 0)),
            full((B, 128)),
            full((4 * EMBED, 64)),
            full((1, 64)),
            full((1, 64)),
            full((64, 16)),
            full((1, 16)),
            full((16, 1)),
            full((1, 1)),
            full(((G + 1) * EMBED, 200)),
            full((1, 200)),
            full((1, 200)),
            full((200, 80)),
            full((1, 80)),
            full((1, 80)),
            full((80, 1)),
            full((1, 1)),
        ],
        out_specs=pl.BlockSpec((B, 1), lambda f, gid: (0, 0)),
        scratch_shapes=[
            pltpu.VMEM((G, B, EMBED), jnp.float32),
            pltpu.VMEM((B, 64), jnp.float32),
        ],
    )
    return pl.pallas_call(
        _tc_body,
        grid_spec=grid_spec,
        out_shape=jax.ShapeDtypeStruct((B, 1), jnp.float32),
    )(gids, ue, le, W1, b1.reshape(1, -1), alpha1.reshape(1, -1),
      W2, b2.reshape(1, -1), W3, b3.reshape(1, -1),
      Wf1, bf1.reshape(1, -1), af1.reshape(1, -1),
      Wf2, bf2.reshape(1, -1), af2.reshape(1, -1),
      Wf3, bf3.reshape(1, -1))


def kernel(batch_user, batch_label, table, W1, b1, alpha1, W2, b2, W3, b3,
           Wf1, bf1, af1, Wf2, bf2, af2, Wf3, bf3):
    idx_user = batch_user.astype(jnp.int32).T.reshape(NW, NCHUNK, CHUNK)
    idx_label = batch_label.astype(jnp.int32).reshape(-1)

    tpad = _transpose_pad_tc(table.T, jnp.eye(EMBED, dtype=jnp.float32))
    ue_flat, le = _gather_sc(tpad, idx_user, idx_label)
    ue = ue_flat.reshape(F, B, 128)

    gid_list = []
    for g, n in enumerate(FEATURE_GROUPS):
        gid_list += [g] * n
    gids = jnp.asarray(gid_list, dtype=jnp.int32)

    return _tc_forward(ue, le, gids, W1, b1, alpha1, W2, b2, W3, b3,
                       Wf1, bf1, af1, Wf2, bf2, af2, Wf3, bf3)


# speedup vs baseline: 3.1731x; 1.0316x over previous
import functools

import jax
import jax.numpy as jnp
from jax import lax
from jax.experimental import pallas as pl
from jax.experimental.pallas import tpu as pltpu
from jax.experimental.pallas import tpu_sc as plsc

ITEM_NUM = 1000000
EMBED = 96
FEATURE_GROUPS = [20, 20, 10, 10, 2, 2, 2, 1, 1, 1]
F = sum(FEATURE_GROUPS)
G = len(FEATURE_GROUPS)
B = 1024

NC, NS = 2, 16
NW = NC * NS
ROWS_W = (B * F) // NW
CHUNK = 96
NCHUNK = ROWS_W // CHUNK
LROWS = B // NW

BN_S = 0.9999950000374997


TBL = 2048
NTB = -(-(ITEM_NUM + 1) // TBL)
TROWS = NTB * TBL


def _transpose_pad_tc(tableT, eye):
    def body(tT_ref, eye_ref, out_ref):
        x = tT_ref[...]
        xT = x.T
        out_ref[:, :EMBED] = xT
        out_ref[:, EMBED:] = jnp.zeros((TBL, 128 - EMBED), jnp.float32)

    return pl.pallas_call(
        body,
        grid=(NTB,),
        in_specs=[pl.BlockSpec((EMBED, TBL), lambda t: (0, t)),
                  pl.BlockSpec((EMBED, EMBED), lambda t: (0, 0))],
        out_specs=pl.BlockSpec((TBL, 128), lambda t: (t, 0)),
        out_shape=jax.ShapeDtypeStruct((TROWS, 128), jnp.float32),
    )(tableT, eye)


def _gather_sc(table, idx_user, idx_label):
    mesh = plsc.VectorSubcoreMesh(core_axis_name="c", subcore_axis_name="s")

    @functools.partial(
        pl.kernel,
        mesh=mesh,
        out_type=[
            jax.ShapeDtypeStruct((B * F, 128), jnp.float32),
            jax.ShapeDtypeStruct((B, 128), jnp.float32),
        ],
        scratch_types=[
            pltpu.VMEM((NCHUNK, CHUNK), jnp.int32),
            pltpu.VMEM((CHUNK, 128), jnp.float32),
            pltpu.VMEM((CHUNK, 128), jnp.float32),
            pltpu.VMEM((LROWS,), jnp.int32),
            pltpu.VMEM((LROWS, 128), jnp.float32),
            pltpu.SemaphoreType.DMA,
            pltpu.SemaphoreType.DMA,
        ],
    )
    def k(table_hbm, idxu_hbm, idxl_hbm, out_u, out_l,
          idx_v, buf0, buf1, idxl_v, lbuf, gsem, wsem):
        wid = lax.axis_index("s") * NC + lax.axis_index("c")
        base = wid * ROWS_W

        pltpu.sync_copy(idxl_hbm.at[pl.ds(wid * LROWS, LROWS)], idxl_v)
        pltpu.async_copy(table_hbm.at[idxl_v], lbuf, gsem).wait()
        pltpu.sync_copy(lbuf, out_l.at[pl.ds(wid * LROWS, LROWS)])

        pltpu.sync_copy(idxu_hbm.at[wid], idx_v)
        bufs = (buf0, buf1)
        pending = [None, None]
        for c in range(NCHUNK):
            b = bufs[c % 2]
            if pending[c % 2] is not None:
                pending[c % 2].wait()
            pltpu.async_copy(table_hbm.at[idx_v.at[c]], b, gsem).wait()
            wb = pltpu.async_copy(
                b, out_u.at[pl.ds(base + c * CHUNK, CHUNK)], wsem)
            pending[c % 2] = wb
        pending[0].wait()
        pending[1].wait()

    return k(table, idx_user, idx_label)


def _dice(x, alpha):
    xp = 1.0 / (1.0 + jnp.exp(-x))
    return alpha * (1.0 - xp) * x + xp * x


def _tc_body(gid_ref, ue_ref, le_ref, W1_ref, b1_ref, alpha1_ref,
             W2_ref, b2_ref, W3_ref, b3_ref,
             Wf1_ref, bf1_ref, af1_ref, Wf2_ref, bf2_ref, af2_ref,
             Wf3_ref, bf3_ref, out_ref, acc_ref, aq_ref):
    f = pl.program_id(0)
    le = le_ref[...][:, :EMBED]
    u = ue_ref[0][:, :EMBED]
    W1 = W1_ref[...]

    @pl.when(f == 0)
    def _init():
        acc_ref[...] = jnp.zeros_like(acc_ref)
        Wq = W1[0:EMBED] + W1[2 * EMBED:3 * EMBED]
        aq_ref[...] = jnp.dot(le, Wq, preferred_element_type=jnp.float32)

    Wu = W1[EMBED:2 * EMBED] - W1[2 * EMBED:3 * EMBED]
    Wm = W1[3 * EMBED:4 * EMBED]
    h1 = (aq_ref[...]
          + jnp.dot(u, Wu, preferred_element_type=jnp.float32)
          + jnp.dot(le * u, Wm, preferred_element_type=jnp.float32)
          + b1_ref[...])
    h1 = _dice(h1, alpha1_ref[...])
    W23 = jnp.dot(W2_ref[...], W3_ref[...],
                  preferred_element_type=jnp.float32)
    c23 = jnp.dot(b2_ref[...], W3_ref[...],
                  preferred_element_type=jnp.float32) + b3_ref[...]
    att = jnp.dot(h1, W23, preferred_element_type=jnp.float32) + c23
    pre = u * att

    g = gid_ref[f]
    acc_ref[g] = acc_ref[g] + pre

    @pl.when(f == F - 1)
    def _final():
        Wf1 = Wf1_ref[...]
        h = jnp.dot(le, Wf1[G * EMBED:], preferred_element_type=jnp.float32)
        for g2 in range(G):
            h = h + jnp.dot(acc_ref[g2], Wf1[g2 * EMBED:(g2 + 1) * EMBED],
                            preferred_element_type=jnp.float32)
        h = (h + bf1_ref[...]) * BN_S
        h = _dice(h, af1_ref[...])
        h = (jnp.dot(h, Wf2_ref[...], preferred_element_type=jnp.float32)
             + bf2_ref[...]) * BN_S
        h = _dice(h, af2_ref[...])
        out_ref[...] = (jnp.dot(h, Wf3_ref[...],
                                preferred_element_type=jnp.float32)
                        + bf3_ref[...])


def _tc_forward(ue, le, gids, W1, b1, alpha1, W2, b2, W3, b3,
                Wf1, bf1, af1, Wf2, bf2, af2, Wf3, bf3):
    def full(shape):
        return pl.BlockSpec(shape, lambda f, gid: (0,) * len(shape))
    grid_spec = pltpu.PrefetchScalarGridSpec(
        num_scalar_prefetch=1,
        grid=(F,),
        in_specs=[
            pl.BlockSpec((1, B, 128), lambda f, gid: (f, 0, 0)),
            full((B, 128)),
            full((4 * EMBED, 64)),
            full((1, 64)),
            full((1, 64)),
            full((64, 16)),
            full((1, 16)),
            full((16, 1)),
            full((1, 1)),
            full(((G + 1) * EMBED, 200)),
            full((1, 200)),
            full((1, 200)),
            full((200, 80)),
            full((1, 80)),
            full((1, 80)),
            full((80, 1)),
            full((1, 1)),
        ],
        out_specs=pl.BlockSpec((B, 1), lambda f, gid: (0, 0)),
        scratch_shapes=[
            pltpu.VMEM((G, B, EMBED), jnp.float32),
            pltpu.VMEM((B, 64), jnp.float32),
        ],
    )
    return pl.pallas_call(
        _tc_body,
        grid_spec=grid_spec,
        out_shape=jax.ShapeDtypeStruct((B, 1), jnp.float32),
    )(gids, ue, le, W1, b1.reshape(1, -1), alpha1.reshape(1, -1),
      W2, b2.reshape(1, -1), W3, b3.reshape(1, -1),
      Wf1, bf1.reshape(1, -1), af1.reshape(1, -1),
      Wf2, bf2.reshape(1, -1), af2.reshape(1, -1),
      Wf3, bf3.reshape(1, -1))


def kernel(batch_user, batch_label, table, W1, b1, alpha1, W2, b2, W3, b3,
           Wf1, bf1, af1, Wf2, bf2, af2, Wf3, bf3):
    idx_user = batch_user.astype(jnp.int32).T.reshape(NW, NCHUNK, CHUNK)
    idx_label = batch_label.astype(jnp.int32).reshape(-1)

    tpad = _transpose_pad_tc(table.T, jnp.eye(EMBED, dtype=jnp.float32))
    ue_flat, le = _gather_sc(tpad, idx_user, idx_label)
    ue = ue_flat.reshape(F, B, 128)

    gid_list = []
    for g, n in enumerate(FEATURE_GROUPS):
        gid_list += [g] * n
    gids = jnp.asarray(gid_list, dtype=jnp.int32)

    return _tc_forward(ue, le, gids, W1, b1, alpha1, W2, b2, W3, b3,
                       Wf1, bf1, af1, Wf2, bf2, af2, Wf3, bf3)


# speedup vs baseline: 4.0369x; 1.2722x over previous
import functools

import jax
import jax.numpy as jnp
from jax import lax
from jax.experimental import pallas as pl
from jax.experimental.pallas import tpu as pltpu
from jax.experimental.pallas import tpu_sc as plsc

ITEM_NUM = 1000000
EMBED = 96
FEATURE_GROUPS = [20, 20, 10, 10, 2, 2, 2, 1, 1, 1]
F = sum(FEATURE_GROUPS)
G = len(FEATURE_GROUPS)
B = 1024

NC, NS = 2, 16
NW = NC * NS
ROWS_W = (B * F) // NW
CHUNK = 96
NCHUNK = ROWS_W // CHUNK
LROWS = B // NW

BN_S = 0.9999950000374997


TBL = 4096
NTB = -(-(ITEM_NUM + 1) // TBL)
TROWS = NTB * TBL


def _transpose_pad_tc(tableT, eye):
    def body(tT_ref, eye_ref, out_ref):
        x = tT_ref[...]
        xT = x.T
        out_ref[:, :EMBED] = xT
        out_ref[:, EMBED:] = jnp.zeros((TBL, 128 - EMBED), jnp.float32)

    return pl.pallas_call(
        body,
        grid=(NTB,),
        in_specs=[pl.BlockSpec((EMBED, TBL), lambda t: (0, t)),
                  pl.BlockSpec((EMBED, EMBED), lambda t: (0, 0))],
        out_specs=pl.BlockSpec((TBL, 128), lambda t: (t, 0)),
        out_shape=jax.ShapeDtypeStruct((TROWS, 128), jnp.float32),
        compiler_params=pltpu.CompilerParams(
            dimension_semantics=("parallel",)),
    )(tableT, eye)


def _gather_sc(table, idx_user, idx_label):
    mesh = plsc.VectorSubcoreMesh(core_axis_name="c", subcore_axis_name="s")

    @functools.partial(
        pl.kernel,
        mesh=mesh,
        out_type=[
            jax.ShapeDtypeStruct((B * F, 128), jnp.float32),
            jax.ShapeDtypeStruct((B, 128), jnp.float32),
        ],
        scratch_types=[
            pltpu.VMEM((NCHUNK, CHUNK), jnp.int32),
            pltpu.VMEM((CHUNK, 128), jnp.float32),
            pltpu.VMEM((CHUNK, 128), jnp.float32),
            pltpu.VMEM((LROWS,), jnp.int32),
            pltpu.VMEM((LROWS, 128), jnp.float32),
            pltpu.SemaphoreType.DMA,
            pltpu.SemaphoreType.DMA,
        ],
    )
    def k(table_hbm, idxu_hbm, idxl_hbm, out_u, out_l,
          idx_v, buf0, buf1, idxl_v, lbuf, gsem, wsem):
        wid = lax.axis_index("s") * NC + lax.axis_index("c")
        base = wid * ROWS_W

        pltpu.sync_copy(idxl_hbm.at[pl.ds(wid * LROWS, LROWS)], idxl_v)
        pltpu.async_copy(table_hbm.at[idxl_v], lbuf, gsem).wait()
        pltpu.sync_copy(lbuf, out_l.at[pl.ds(wid * LROWS, LROWS)])

        pltpu.sync_copy(idxu_hbm.at[wid], idx_v)
        bufs = (buf0, buf1)
        pending = [None, None]
        for c in range(NCHUNK):
            b = bufs[c % 2]
            if pending[c % 2] is not None:
                pending[c % 2].wait()
            pltpu.async_copy(table_hbm.at[idx_v.at[c]], b, gsem).wait()
            wb = pltpu.async_copy(
                b, out_u.at[pl.ds(base + c * CHUNK, CHUNK)], wsem)
            pending[c % 2] = wb
        pending[0].wait()
        pending[1].wait()

    return k(table, idx_user, idx_label)


def _dice(x, alpha):
    xp = 1.0 / (1.0 + jnp.exp(-x))
    return alpha * (1.0 - xp) * x + xp * x


def _tc_body(gid_ref, ue_ref, le_ref, W1_ref, b1_ref, alpha1_ref,
             W2_ref, b2_ref, W3_ref, b3_ref,
             Wf1_ref, bf1_ref, af1_ref, Wf2_ref, bf2_ref, af2_ref,
             Wf3_ref, bf3_ref, out_ref, acc_ref, aq_ref):
    f = pl.program_id(0)
    le = le_ref[...][:, :EMBED]
    u = ue_ref[0][:, :EMBED]
    W1 = W1_ref[...]

    @pl.when(f == 0)
    def _init():
        acc_ref[...] = jnp.zeros_like(acc_ref)
        Wq = W1[0:EMBED] + W1[2 * EMBED:3 * EMBED]
        aq_ref[...] = jnp.dot(le, Wq, preferred_element_type=jnp.float32)

    Wu = W1[EMBED:2 * EMBED] - W1[2 * EMBED:3 * EMBED]
    Wm = W1[3 * EMBED:4 * EMBED]
    h1 = (aq_ref[...]
          + jnp.dot(u, Wu, preferred_element_type=jnp.float32)
          + jnp.dot(le * u, Wm, preferred_element_type=jnp.float32)
          + b1_ref[...])
    h1 = _dice(h1, alpha1_ref[...])
    W23 = jnp.dot(W2_ref[...], W3_ref[...],
                  preferred_element_type=jnp.float32)
    c23 = jnp.dot(b2_ref[...], W3_ref[...],
                  preferred_element_type=jnp.float32) + b3_ref[...]
    att = jnp.dot(h1, W23, preferred_element_type=jnp.float32) + c23
    pre = u * att

    g = gid_ref[f]
    acc_ref[g] = acc_ref[g] + pre

    @pl.when(f == F - 1)
    def _final():
        Wf1 = Wf1_ref[...]
        h = jnp.dot(le, Wf1[G * EMBED:], preferred_element_type=jnp.float32)
        for g2 in range(G):
            h = h + jnp.dot(acc_ref[g2], Wf1[g2 * EMBED:(g2 + 1) * EMBED],
                            preferred_element_type=jnp.float32)
        h = (h + bf1_ref[...]) * BN_S
        h = _dice(h, af1_ref[...])
        h = (jnp.dot(h, Wf2_ref[...], preferred_element_type=jnp.float32)
             + bf2_ref[...]) * BN_S
        h = _dice(h, af2_ref[...])
        out_ref[...] = (jnp.dot(h, Wf3_ref[...],
                                preferred_element_type=jnp.float32)
                        + bf3_ref[...])


def _tc_forward(ue, le, gids, W1, b1, alpha1, W2, b2, W3, b3,
                Wf1, bf1, af1, Wf2, bf2, af2, Wf3, bf3):
    def full(shape):
        return pl.BlockSpec(shape, lambda f, gid: (0,) * len(shape))
    grid_spec = pltpu.PrefetchScalarGridSpec(
        num_scalar_prefetch=1,
        grid=(F,),
        in_specs=[
            pl.BlockSpec((1, B, 128), lambda f, gid: (f, 0, 0)),
            full((B, 128)),
            full((4 * EMBED, 64)),
            full((1, 64)),
            full((1, 64)),
            full((64, 16)),
            full((1, 16)),
            full((16, 1)),
            full((1, 1)),
            full(((G + 1) * EMBED, 200)),
            full((1, 200)),
            full((1, 200)),
            full((200, 80)),
            full((1, 80)),
            full((1, 80)),
            full((80, 1)),
            full((1, 1)),
        ],
        out_specs=pl.BlockSpec((B, 1), lambda f, gid: (0, 0)),
        scratch_shapes=[
            pltpu.VMEM((G, B, EMBED), jnp.float32),
            pltpu.VMEM((B, 64), jnp.float32),
        ],
    )
    return pl.pallas_call(
        _tc_body,
        grid_spec=grid_spec,
        out_shape=jax.ShapeDtypeStruct((B, 1), jnp.float32),
    )(gids, ue, le, W1, b1.reshape(1, -1), alpha1.reshape(1, -1),
      W2, b2.reshape(1, -1), W3, b3.reshape(1, -1),
      Wf1, bf1.reshape(1, -1), af1.reshape(1, -1),
      Wf2, bf2.reshape(1, -1), af2.reshape(1, -1),
      Wf3, bf3.reshape(1, -1))


def kernel(batch_user, batch_label, table, W1, b1, alpha1, W2, b2, W3, b3,
           Wf1, bf1, af1, Wf2, bf2, af2, Wf3, bf3):
    idx_user = batch_user.astype(jnp.int32).T.reshape(NW, NCHUNK, CHUNK)
    idx_label = batch_label.astype(jnp.int32).reshape(-1)

    tpad = _transpose_pad_tc(table.T, jnp.eye(EMBED, dtype=jnp.float32))
    ue_flat, le = _gather_sc(tpad, idx_user, idx_label)
    ue = ue_flat.reshape(F, B, 128)

    gid_list = []
    for g, n in enumerate(FEATURE_GROUPS):
        gid_list += [g] * n
    gids = jnp.asarray(gid_list, dtype=jnp.int32)

    return _tc_forward(ue, le, gids, W1, b1, alpha1, W2, b2, W3, b3,
                       Wf1, bf1, af1, Wf2, bf2, af2, Wf3, bf3)


# speedup vs baseline: 4.0785x; 1.0103x over previous
import functools

import jax
import jax.numpy as jnp
from jax import lax
from jax.experimental import pallas as pl
from jax.experimental.pallas import tpu as pltpu
from jax.experimental.pallas import tpu_sc as plsc

ITEM_NUM = 1000000
EMBED = 96
FEATURE_GROUPS = [20, 20, 10, 10, 2, 2, 2, 1, 1, 1]
F = sum(FEATURE_GROUPS)
G = len(FEATURE_GROUPS)
B = 1024

NC, NS = 2, 16
NW = NC * NS
ROWS_W = (B * F) // NW
CHUNK = 96
NCHUNK = ROWS_W // CHUNK
LROWS = B // NW

BN_S = 0.9999950000374997


TBL = 4096
NTB = -(-(ITEM_NUM + 1) // TBL)
TROWS = NTB * TBL


def _transpose_pad_tc(tableT, eye):
    def body(tT_ref, eye_ref, out_ref):
        x = tT_ref[...]
        xT = x.T
        out_ref[:, :EMBED] = xT
        out_ref[:, EMBED:] = jnp.zeros((TBL, 128 - EMBED), jnp.float32)

    return pl.pallas_call(
        body,
        grid=(NTB,),
        in_specs=[pl.BlockSpec((EMBED, TBL), lambda t: (0, t)),
                  pl.BlockSpec((EMBED, EMBED), lambda t: (0, 0))],
        out_specs=pl.BlockSpec((TBL, 128), lambda t: (t, 0)),
        out_shape=jax.ShapeDtypeStruct((TROWS, 128), jnp.float32),
        compiler_params=pltpu.CompilerParams(
            dimension_semantics=("parallel",)),
    )(tableT, eye)


def _gather_sc(table, idx_user, idx_label):
    mesh = plsc.VectorSubcoreMesh(core_axis_name="c", subcore_axis_name="s")

    @functools.partial(
        pl.kernel,
        mesh=mesh,
        out_type=[
            jax.ShapeDtypeStruct((B * F, 128), jnp.float32),
            jax.ShapeDtypeStruct((B, 128), jnp.float32),
        ],
        scratch_types=[
            pltpu.VMEM((NCHUNK, CHUNK), jnp.int32),
            pltpu.VMEM((CHUNK, 128), jnp.float32),
            pltpu.VMEM((CHUNK, 128), jnp.float32),
            pltpu.VMEM((LROWS,), jnp.int32),
            pltpu.VMEM((LROWS, 128), jnp.float32),
            pltpu.SemaphoreType.DMA,
            pltpu.SemaphoreType.DMA,
        ],
    )
    def k(table_hbm, idxu_hbm, idxl_hbm, out_u, out_l,
          idx_v, buf0, buf1, idxl_v, lbuf, gsem, wsem):
        wid = lax.axis_index("s") * NC + lax.axis_index("c")
        base = wid * ROWS_W

        pltpu.sync_copy(idxl_hbm.at[pl.ds(wid * LROWS, LROWS)], idxl_v)
        pltpu.async_copy(table_hbm.at[idxl_v], lbuf, gsem).wait()
        pltpu.sync_copy(lbuf, out_l.at[pl.ds(wid * LROWS, LROWS)])

        pltpu.sync_copy(idxu_hbm.at[wid], idx_v)
        bufs = (buf0, buf1)
        pending = [None, None]
        for c in range(NCHUNK):
            b = bufs[c % 2]
            if pending[c % 2] is not None:
                pending[c % 2].wait()
            pltpu.async_copy(table_hbm.at[idx_v.at[c]], b, gsem).wait()
            wb = pltpu.async_copy(
                b, out_u.at[pl.ds(base + c * CHUNK, CHUNK)], wsem)
            pending[c % 2] = wb
        pending[0].wait()
        pending[1].wait()

    return k(table, idx_user, idx_label)


def _dice(x, alpha):
    xp = 1.0 / (1.0 + jnp.exp(-x))
    return alpha * (1.0 - xp) * x + xp * x


def _tc_body(ue_hbm, le_ref, W1_ref, b1_ref, alpha1_ref,
             W2_ref, b2_ref, W3_ref, b3_ref,
             Wf1_ref, bf1_ref, af1_ref, Wf2_ref, bf2_ref, af2_ref,
             Wf3_ref, bf3_ref, out_ref, ubuf, sems):
    def ucopy(f):
        return pltpu.make_async_copy(ue_hbm.at[f], ubuf.at[f % 2],
                                     sems.at[f % 2])

    def dot(a, b):
        return jnp.dot(a, b, preferred_element_type=jnp.float32)

    le = le_ref[...][:, :EMBED]
    W1 = W1_ref[...]
    Wq = W1[0:EMBED] + W1[2 * EMBED:3 * EMBED]
    Wum = jnp.concatenate([W1[EMBED:2 * EMBED] - W1[2 * EMBED:3 * EMBED],
                           W1[3 * EMBED:4 * EMBED]], axis=0)
    aq = dot(le, Wq) + b1_ref[...]
    alpha1 = alpha1_ref[...]
    W23 = dot(W2_ref[...], W3_ref[...])
    c23 = dot(b2_ref[...], W3_ref[...]) + b3_ref[...]

    ucopy(0).start()
    parts = [le]
    f = 0
    for n in FEATURE_GROUPS:
        psum = None
        for _ in range(n):
            if f + 1 < F:
                ucopy(f + 1).start()
            ucopy(f).wait()
            u = ubuf[f % 2][:, :EMBED]
            h1 = _dice(aq + dot(jnp.concatenate([u, le * u], axis=1), Wum),
                       alpha1)
            att = dot(h1, W23) + c23
            pre = u * att
            psum = pre if psum is None else psum + pre
            f += 1
        parts.append(psum)
    x = jnp.concatenate(parts[1:] + parts[:1], axis=1)
    h = (dot(x, Wf1_ref[...]) + bf1_ref[...]) * BN_S
    h = _dice(h, af1_ref[...])
    h = (dot(h, Wf2_ref[...]) + bf2_ref[...]) * BN_S
    h = _dice(h, af2_ref[...])
    out_ref[...] = dot(h, Wf3_ref[...]) + bf3_ref[...]


def _tc_forward(ue, le, W1, b1, alpha1, W2, b2, W3, b3,
                Wf1, bf1, af1, Wf2, bf2, af2, Wf3, bf3):
    args = (ue, le, W1, b1.reshape(1, -1), alpha1.reshape(1, -1),
            W2, b2.reshape(1, -1), W3, b3.reshape(1, -1),
            Wf1, bf1.reshape(1, -1), af1.reshape(1, -1),
            Wf2, bf2.reshape(1, -1), af2.reshape(1, -1),
            Wf3, bf3.reshape(1, -1))
    return pl.pallas_call(
        _tc_body,
        in_specs=[pl.BlockSpec(memory_space=pl.ANY)]
                 + [pl.BlockSpec(memory_space=pltpu.MemorySpace.VMEM)] * 16,
        out_specs=pl.BlockSpec(memory_space=pltpu.MemorySpace.VMEM),
        out_shape=jax.ShapeDtypeStruct((B, 1), jnp.float32),
        scratch_shapes=[pltpu.VMEM((2, B, 128), jnp.float32),
                        pltpu.SemaphoreType.DMA((2,))],
    )(*args)


def kernel(batch_user, batch_label, table, W1, b1, alpha1, W2, b2, W3, b3,
           Wf1, bf1, af1, Wf2, bf2, af2, Wf3, bf3):
    idx_user = batch_user.astype(jnp.int32).T.reshape(NW, NCHUNK, CHUNK)
    idx_label = batch_label.astype(jnp.int32).reshape(-1)

    tpad = _transpose_pad_tc(table.T, jnp.eye(EMBED, dtype=jnp.float32))
    ue_flat, le = _gather_sc(tpad, idx_user, idx_label)
    ue = ue_flat.reshape(F, B, 128)

    return _tc_forward(ue, le, W1, b1, alpha1, W2, b2, W3, b3,
                       Wf1, bf1, af1, Wf2, bf2, af2, Wf3, bf3)


# speedup vs baseline: 4.7193x; 1.1571x over previous
import functools

import jax
import jax.numpy as jnp
from jax import lax
from jax.experimental import pallas as pl
from jax.experimental.pallas import tpu as pltpu
from jax.experimental.pallas import tpu_sc as plsc

ITEM_NUM = 1000000
EMBED = 96
FEATURE_GROUPS = [20, 20, 10, 10, 2, 2, 2, 1, 1, 1]
F = sum(FEATURE_GROUPS)
G = len(FEATURE_GROUPS)
B = 1024

NC, NS = 2, 16
NW = NC * NS
ROWS_W = (B * F) // NW
CHUNK = 96
NCHUNK = ROWS_W // CHUNK
LROWS = B // NW

BN_S = 0.9999950000374997


TBL = 8192
NTB = -(-(ITEM_NUM + 1) // TBL)
TROWS = NTB * TBL


def _transpose_pad_tc(tableT, eye):
    def body(tT_ref, eye_ref, out_ref):
        x = tT_ref[...]
        xT = x.T
        out_ref[:, :EMBED] = xT
        out_ref[:, EMBED:] = jnp.zeros((TBL, 128 - EMBED), jnp.float32)

    return pl.pallas_call(
        body,
        grid=(NTB,),
        in_specs=[pl.BlockSpec((EMBED, TBL), lambda t: (0, t)),
                  pl.BlockSpec((EMBED, EMBED), lambda t: (0, 0))],
        out_specs=pl.BlockSpec((TBL, 128), lambda t: (t, 0)),
        out_shape=jax.ShapeDtypeStruct((TROWS, 128), jnp.float32),
        compiler_params=pltpu.CompilerParams(
            dimension_semantics=("parallel",)),
    )(tableT, eye)


def _gather_sc(table, idx_user, idx_label):
    mesh = plsc.VectorSubcoreMesh(core_axis_name="c", subcore_axis_name="s")

    @functools.partial(
        pl.kernel,
        mesh=mesh,
        out_type=[
            jax.ShapeDtypeStruct((B * F, 128), jnp.float32),
            jax.ShapeDtypeStruct((B, 128), jnp.float32),
        ],
        scratch_types=[
            pltpu.VMEM((NCHUNK, CHUNK), jnp.int32),
            pltpu.VMEM((CHUNK, 128), jnp.float32),
            pltpu.VMEM((CHUNK, 128), jnp.float32),
            pltpu.VMEM((LROWS,), jnp.int32),
            pltpu.VMEM((LROWS, 128), jnp.float32),
            pltpu.SemaphoreType.DMA,
            pltpu.SemaphoreType.DMA,
        ],
    )
    def k(table_hbm, idxu_hbm, idxl_hbm, out_u, out_l,
          idx_v, buf0, buf1, idxl_v, lbuf, gsem, wsem):
        wid = lax.axis_index("s") * NC + lax.axis_index("c")
        base = wid * ROWS_W

        pltpu.sync_copy(idxl_hbm.at[pl.ds(wid * LROWS, LROWS)], idxl_v)
        pltpu.async_copy(table_hbm.at[idxl_v], lbuf, gsem).wait()
        pltpu.sync_copy(lbuf, out_l.at[pl.ds(wid * LROWS, LROWS)])

        pltpu.sync_copy(idxu_hbm.at[wid], idx_v)
        bufs = (buf0, buf1)
        pending = [None, None]
        for c in range(NCHUNK):
            b = bufs[c % 2]
            if pending[c % 2] is not None:
                pending[c % 2].wait()
            pltpu.async_copy(table_hbm.at[idx_v.at[c]], b, gsem).wait()
            wb = pltpu.async_copy(
                b, out_u.at[pl.ds(base + c * CHUNK, CHUNK)], wsem)
            pending[c % 2] = wb
        pending[0].wait()
        pending[1].wait()

    return k(table, idx_user, idx_label)


def _dice(x, alpha):
    xp = 1.0 / (1.0 + jnp.exp(-x))
    return alpha * (1.0 - xp) * x + xp * x


def _tc_body(ue_hbm, le_ref, W1_ref, b1_ref, alpha1_ref,
             W2_ref, b2_ref, W3_ref, b3_ref,
             Wf1_ref, bf1_ref, af1_ref, Wf2_ref, bf2_ref, af2_ref,
             Wf3_ref, bf3_ref, out_ref, ubuf, sems):
    def ucopy(f):
        return pltpu.make_async_copy(ue_hbm.at[f], ubuf.at[f % 2],
                                     sems.at[f % 2])

    def dot(a, b):
        return jnp.dot(a, b, preferred_element_type=jnp.float32)

    le = le_ref[...][:, :EMBED]
    W1 = W1_ref[...]
    Wq = W1[0:EMBED] + W1[2 * EMBED:3 * EMBED]
    Wum = jnp.concatenate([W1[EMBED:2 * EMBED] - W1[2 * EMBED:3 * EMBED],
                           W1[3 * EMBED:4 * EMBED]], axis=0)
    aq = dot(le, Wq) + b1_ref[...]
    alpha1 = alpha1_ref[...]
    W23 = dot(W2_ref[...], W3_ref[...])
    c23 = dot(b2_ref[...], W3_ref[...]) + b3_ref[...]

    ucopy(0).start()
    parts = [le]
    f = 0
    for n in FEATURE_GROUPS:
        psum = None
        for _ in range(n):
            if f + 1 < F:
                ucopy(f + 1).start()
            ucopy(f).wait()
            u = ubuf[f % 2][:, :EMBED]
            h1 = _dice(aq + dot(jnp.concatenate([u, le * u], axis=1), Wum),
                       alpha1)
            att = dot(h1, W23) + c23
            pre = u * att
            psum = pre if psum is None else psum + pre
            f += 1
        parts.append(psum)
    x = jnp.concatenate(parts[1:] + parts[:1], axis=1)
    h = (dot(x, Wf1_ref[...]) + bf1_ref[...]) * BN_S
    h = _dice(h, af1_ref[...])
    h = (dot(h, Wf2_ref[...]) + bf2_ref[...]) * BN_S
    h = _dice(h, af2_ref[...])
    out_ref[...] = dot(h, Wf3_ref[...]) + bf3_ref[...]


def _tc_forward(ue, le, W1, b1, alpha1, W2, b2, W3, b3,
                Wf1, bf1, af1, Wf2, bf2, af2, Wf3, bf3):
    args = (ue, le, W1, b1.reshape(1, -1), alpha1.reshape(1, -1),
            W2, b2.reshape(1, -1), W3, b3.reshape(1, -1),
            Wf1, bf1.reshape(1, -1), af1.reshape(1, -1),
            Wf2, bf2.reshape(1, -1), af2.reshape(1, -1),
            Wf3, bf3.reshape(1, -1))
    return pl.pallas_call(
        _tc_body,
        in_specs=[pl.BlockSpec(memory_space=pl.ANY)]
                 + [pl.BlockSpec(memory_space=pltpu.MemorySpace.VMEM)] * 16,
        out_specs=pl.BlockSpec(memory_space=pltpu.MemorySpace.VMEM),
        out_shape=jax.ShapeDtypeStruct((B, 1), jnp.float32),
        scratch_shapes=[pltpu.VMEM((2, B, 128), jnp.float32),
                        pltpu.SemaphoreType.DMA((2,))],
    )(*args)


def kernel(batch_user, batch_label, table, W1, b1, alpha1, W2, b2, W3, b3,
           Wf1, bf1, af1, Wf2, bf2, af2, Wf3, bf3):
    idx_user = batch_user.astype(jnp.int32).T.reshape(NW, NCHUNK, CHUNK)
    idx_label = batch_label.astype(jnp.int32).reshape(-1)

    tpad = _transpose_pad_tc(table.T, jnp.eye(EMBED, dtype=jnp.float32))
    ue_flat, le = _gather_sc(tpad, idx_user, idx_label)
    ue = ue_flat.reshape(F, B, 128)

    return _tc_forward(ue, le, W1, b1, alpha1, W2, b2, W3, b3,
                       Wf1, bf1, af1, Wf2, bf2, af2, Wf3, bf3)


# speedup vs baseline: 4.8313x; 1.0237x over previous
import functools

import jax
import jax.numpy as jnp
from jax import lax
from jax.experimental import pallas as pl
from jax.experimental.pallas import tpu as pltpu
from jax.experimental.pallas import tpu_sc as plsc

ITEM_NUM = 1000000
EMBED = 96
FEATURE_GROUPS = [20, 20, 10, 10, 2, 2, 2, 1, 1, 1]
F = sum(FEATURE_GROUPS)
G = len(FEATURE_GROUPS)
B = 1024

NC, NS = 2, 16
NW = NC * NS
ROWS_W = (B * F) // NW
CHUNK = 96
NCHUNK = ROWS_W // CHUNK
LROWS = B // NW

BN_S = 0.9999950000374997


TBL = 16384
NTB = -(-(ITEM_NUM + 1) // TBL)
TROWS = NTB * TBL


def _transpose_pad_tc(tableT, eye):
    def body(tT_ref, eye_ref, out_ref):
        x = tT_ref[...]
        xT = x.T
        out_ref[:, :EMBED] = xT
        out_ref[:, EMBED:] = jnp.zeros((TBL, 128 - EMBED), jnp.float32)

    return pl.pallas_call(
        body,
        grid=(NTB,),
        in_specs=[pl.BlockSpec((EMBED, TBL), lambda t: (0, t)),
                  pl.BlockSpec((EMBED, EMBED), lambda t: (0, 0))],
        out_specs=pl.BlockSpec((TBL, 128), lambda t: (t, 0)),
        out_shape=jax.ShapeDtypeStruct((TROWS, 128), jnp.float32),
        compiler_params=pltpu.CompilerParams(
            dimension_semantics=("parallel",)),
    )(tableT, eye)


def _gather_sc(table, idx_user, idx_label):
    mesh = plsc.VectorSubcoreMesh(core_axis_name="c", subcore_axis_name="s")

    @functools.partial(
        pl.kernel,
        mesh=mesh,
        out_type=[
            jax.ShapeDtypeStruct((B * F, 128), jnp.float32),
            jax.ShapeDtypeStruct((B, 128), jnp.float32),
        ],
        scratch_types=[
            pltpu.VMEM((NCHUNK, CHUNK), jnp.int32),
            pltpu.VMEM((CHUNK, 128), jnp.float32),
            pltpu.VMEM((CHUNK, 128), jnp.float32),
            pltpu.VMEM((LROWS,), jnp.int32),
            pltpu.VMEM((LROWS, 128), jnp.float32),
            pltpu.SemaphoreType.DMA,
            pltpu.SemaphoreType.DMA,
        ],
    )
    def k(table_hbm, idxu_hbm, idxl_hbm, out_u, out_l,
          idx_v, buf0, buf1, idxl_v, lbuf, gsem, wsem):
        wid = lax.axis_index("s") * NC + lax.axis_index("c")
        base = wid * ROWS_W

        pltpu.sync_copy(idxl_hbm.at[pl.ds(wid * LROWS, LROWS)], idxl_v)
        pltpu.async_copy(table_hbm.at[idxl_v], lbuf, gsem).wait()
        pltpu.sync_copy(lbuf, out_l.at[pl.ds(wid * LROWS, LROWS)])

        pltpu.sync_copy(idxu_hbm.at[wid], idx_v)
        bufs = (buf0, buf1)
        pending = [None, None]
        for c in range(NCHUNK):
            b = bufs[c % 2]
            if pending[c % 2] is not None:
                pending[c % 2].wait()
            pltpu.async_copy(table_hbm.at[idx_v.at[c]], b, gsem).wait()
            wb = pltpu.async_copy(
                b, out_u.at[pl.ds(base + c * CHUNK, CHUNK)], wsem)
            pending[c % 2] = wb
        pending[0].wait()
        pending[1].wait()

    return k(table, idx_user, idx_label)


def _dice(x, alpha):
    xp = 1.0 / (1.0 + jnp.exp(-x))
    return alpha * (1.0 - xp) * x + xp * x


def _tc_body(ue_hbm, le_ref, W1_ref, b1_ref, alpha1_ref,
             W2_ref, b2_ref, W3_ref, b3_ref,
             Wf1_ref, bf1_ref, af1_ref, Wf2_ref, bf2_ref, af2_ref,
             Wf3_ref, bf3_ref, out_ref, ubuf, sems):
    def ucopy(f):
        return pltpu.make_async_copy(ue_hbm.at[f], ubuf.at[f % 2],
                                     sems.at[f % 2])

    def dot(a, b):
        return jnp.dot(a, b, preferred_element_type=jnp.float32)

    le = le_ref[...][:, :EMBED]
    W1 = W1_ref[...]
    Wq = W1[0:EMBED] + W1[2 * EMBED:3 * EMBED]
    Wum = jnp.concatenate([W1[EMBED:2 * EMBED] - W1[2 * EMBED:3 * EMBED],
                           W1[3 * EMBED:4 * EMBED]], axis=0)
    aq = dot(le, Wq) + b1_ref[...]
    alpha1 = alpha1_ref[...]
    W23 = dot(W2_ref[...], W3_ref[...])
    c23 = dot(b2_ref[...], W3_ref[...]) + b3_ref[...]

    ucopy(0).start()
    parts = [le]
    f = 0
    for n in FEATURE_GROUPS:
        psum = None
        for _ in range(n):
            if f + 1 < F:
                ucopy(f + 1).start()
            ucopy(f).wait()
            u = ubuf[f % 2][:, :EMBED]
            h1 = _dice(aq + dot(jnp.concatenate([u, le * u], axis=1), Wum),
                       alpha1)
            att = dot(h1, W23) + c23
            pre = u * att
            psum = pre if psum is None else psum + pre
            f += 1
        parts.append(psum)
    x = jnp.concatenate(parts[1:] + parts[:1], axis=1)
    h = (dot(x, Wf1_ref[...]) + bf1_ref[...]) * BN_S
    h = _dice(h, af1_ref[...])
    h = (dot(h, Wf2_ref[...]) + bf2_ref[...]) * BN_S
    h = _dice(h, af2_ref[...])
    out_ref[...] = dot(h, Wf3_ref[...]) + bf3_ref[...]


def _tc_forward(ue, le, W1, b1, alpha1, W2, b2, W3, b3,
                Wf1, bf1, af1, Wf2, bf2, af2, Wf3, bf3):
    args = (ue, le, W1, b1.reshape(1, -1), alpha1.reshape(1, -1),
            W2, b2.reshape(1, -1), W3, b3.reshape(1, -1),
            Wf1, bf1.reshape(1, -1), af1.reshape(1, -1),
            Wf2, bf2.reshape(1, -1), af2.reshape(1, -1),
            Wf3, bf3.reshape(1, -1))
    return pl.pallas_call(
        _tc_body,
        in_specs=[pl.BlockSpec(memory_space=pl.ANY)]
                 + [pl.BlockSpec(memory_space=pltpu.MemorySpace.VMEM)] * 16,
        out_specs=pl.BlockSpec(memory_space=pltpu.MemorySpace.VMEM),
        out_shape=jax.ShapeDtypeStruct((B, 1), jnp.float32),
        scratch_shapes=[pltpu.VMEM((2, B, 128), jnp.float32),
                        pltpu.SemaphoreType.DMA((2,))],
    )(*args)


def kernel(batch_user, batch_label, table, W1, b1, alpha1, W2, b2, W3, b3,
           Wf1, bf1, af1, Wf2, bf2, af2, Wf3, bf3):
    idx_user = batch_user.astype(jnp.int32).T.reshape(NW, NCHUNK, CHUNK)
    idx_label = batch_label.astype(jnp.int32).reshape(-1)

    tpad = _transpose_pad_tc(table.T, jnp.eye(EMBED, dtype=jnp.float32))
    ue_flat, le = _gather_sc(tpad, idx_user, idx_label)
    ue = ue_flat.reshape(F, B, 128)

    return _tc_forward(ue, le, W1, b1, alpha1, W2, b2, W3, b3,
                       Wf1, bf1, af1, Wf2, bf2, af2, Wf3, bf3)


# speedup vs baseline: 6.1540x; 1.2738x over previous
import functools

import jax
import jax.numpy as jnp
from jax import lax
from jax.experimental import pallas as pl
from jax.experimental.pallas import tpu as pltpu
from jax.experimental.pallas import tpu_sc as plsc

ITEM_NUM = 1000000
EMBED = 96
FEATURE_GROUPS = [20, 20, 10, 10, 2, 2, 2, 1, 1, 1]
F = sum(FEATURE_GROUPS)
G = len(FEATURE_GROUPS)
B = 1024

NC, NS = 2, 16
NW = NC * NS
ROWS_W = (B * F) // NW
CHUNK = 96
NCHUNK = ROWS_W // CHUNK
LROWS = B // NW

BN_S = 0.9999950000374997


TBL = 16384
NTB = -(-(ITEM_NUM + 1) // TBL)
TROWS = NTB * TBL


def _transpose_pad_tc(tableT, eye):
    def body(tT_ref, eye_ref, out_ref):
        x = tT_ref[...]
        xT = x.T
        y = xT.astype(jnp.bfloat16)
        pf = pltpu.bitcast(y, jnp.float32)
        out_ref[:, :EMBED] = pf
        out_ref[:, EMBED:] = jnp.zeros((TBL // 2, 128 - EMBED), jnp.float32)

    return pl.pallas_call(
        body,
        grid=(NTB,),
        in_specs=[pl.BlockSpec((EMBED, TBL), lambda t: (0, t)),
                  pl.BlockSpec((EMBED, EMBED), lambda t: (0, 0))],
        out_specs=pl.BlockSpec((TBL // 2, 128), lambda t: (t, 0)),
        out_shape=jax.ShapeDtypeStruct((TROWS // 2, 128), jnp.float32),
        compiler_params=pltpu.CompilerParams(
            dimension_semantics=("parallel",)),
    )(tableT, eye)


def _gather_sc(table, idx_user, idx_label):
    mesh = plsc.VectorSubcoreMesh(core_axis_name="c", subcore_axis_name="s")

    @functools.partial(
        pl.kernel,
        mesh=mesh,
        out_type=[
            jax.ShapeDtypeStruct((B * F, 128), jnp.float32),
            jax.ShapeDtypeStruct((B, 128), jnp.float32),
        ],
        scratch_types=[
            pltpu.VMEM((NCHUNK, CHUNK), jnp.int32),
            pltpu.VMEM((CHUNK, 128), jnp.float32),
            pltpu.VMEM((CHUNK, 128), jnp.float32),
            pltpu.VMEM((LROWS,), jnp.int32),
            pltpu.VMEM((LROWS, 128), jnp.float32),
            pltpu.SemaphoreType.DMA,
            pltpu.SemaphoreType.DMA,
        ],
    )
    def k(table_hbm, idxu_hbm, idxl_hbm, out_u, out_l,
          idx_v, buf0, buf1, idxl_v, lbuf, gsem, wsem):
        wid = lax.axis_index("s") * NC + lax.axis_index("c")
        base = wid * ROWS_W

        pltpu.sync_copy(idxl_hbm.at[pl.ds(wid * LROWS, LROWS)], idxl_v)
        pltpu.async_copy(table_hbm.at[idxl_v], lbuf, gsem).wait()
        pltpu.sync_copy(lbuf, out_l.at[pl.ds(wid * LROWS, LROWS)])

        pltpu.sync_copy(idxu_hbm.at[wid], idx_v)
        bufs = (buf0, buf1)
        pending = [None, None]
        for c in range(NCHUNK):
            b = bufs[c % 2]
            if pending[c % 2] is not None:
                pending[c % 2].wait()
            pltpu.async_copy(table_hbm.at[idx_v.at[c]], b, gsem).wait()
            wb = pltpu.async_copy(
                b, out_u.at[pl.ds(base + c * CHUNK, CHUNK)], wsem)
            pending[c % 2] = wb
        pending[0].wait()
        pending[1].wait()

    return k(table, idx_user, idx_label)


def _dice(x, alpha):
    xp = 1.0 / (1.0 + jnp.exp(-x))
    return alpha * (1.0 - xp) * x + xp * x


def _tc_body(ue_hbm, le_ref, par_ref, W1_ref, b1_ref, alpha1_ref,
             W2_ref, b2_ref, W3_ref, b3_ref,
             Wf1_ref, bf1_ref, af1_ref, Wf2_ref, bf2_ref, af2_ref,
             Wf3_ref, bf3_ref, out_ref, ubuf, sems):
    def ucopy(f):
        return pltpu.make_async_copy(ue_hbm.at[f], ubuf.at[f % 2],
                                     sems.at[f % 2])

    def unpack(raw, par):
        wu = jax.lax.bitcast_convert_type(raw, jnp.uint32)
        lo = jax.lax.bitcast_convert_type(wu << 16, jnp.float32)
        hi = jax.lax.bitcast_convert_type(wu & jnp.uint32(0xFFFF0000),
                                          jnp.float32)
        return jnp.where(par > 0.5, hi, lo)

    def dot(a, b):
        return jnp.dot(a, b, preferred_element_type=jnp.float32)

    le = unpack(le_ref[...][:, :EMBED], par_ref[:, F:F + 1])
    W1 = W1_ref[...]
    Wq = W1[0:EMBED] + W1[2 * EMBED:3 * EMBED]
    Wum = jnp.concatenate([W1[EMBED:2 * EMBED] - W1[2 * EMBED:3 * EMBED],
                           W1[3 * EMBED:4 * EMBED]], axis=0)
    aq = dot(le, Wq) + b1_ref[...]
    alpha1 = alpha1_ref[...]
    W23 = dot(W2_ref[...], W3_ref[...])
    c23 = dot(b2_ref[...], W3_ref[...]) + b3_ref[...]

    ucopy(0).start()
    parts = [le]
    f = 0
    for n in FEATURE_GROUPS:
        psum = None
        for _ in range(n):
            if f + 1 < F:
                ucopy(f + 1).start()
            ucopy(f).wait()
            u = unpack(ubuf[f % 2][:, :EMBED], par_ref[:, f:f + 1])
            h1 = _dice(aq + dot(jnp.concatenate([u, le * u], axis=1), Wum),
                       alpha1)
            att = dot(h1, W23) + c23
            pre = u * att
            psum = pre if psum is None else psum + pre
            f += 1
        parts.append(psum)
    x = jnp.concatenate(parts[1:] + parts[:1], axis=1)
    h = (dot(x, Wf1_ref[...]) + bf1_ref[...]) * BN_S
    h = _dice(h, af1_ref[...])
    h = (dot(h, Wf2_ref[...]) + bf2_ref[...]) * BN_S
    h = _dice(h, af2_ref[...])
    out_ref[...] = dot(h, Wf3_ref[...]) + bf3_ref[...]


def _tc_forward(ue, le, par, W1, b1, alpha1, W2, b2, W3, b3,
                Wf1, bf1, af1, Wf2, bf2, af2, Wf3, bf3):
    args = (ue, le, par, W1, b1.reshape(1, -1), alpha1.reshape(1, -1),
            W2, b2.reshape(1, -1), W3, b3.reshape(1, -1),
            Wf1, bf1.reshape(1, -1), af1.reshape(1, -1),
            Wf2, bf2.reshape(1, -1), af2.reshape(1, -1),
            Wf3, bf3.reshape(1, -1))
    return pl.pallas_call(
        _tc_body,
        in_specs=[pl.BlockSpec(memory_space=pl.ANY)]
                 + [pl.BlockSpec(memory_space=pltpu.MemorySpace.VMEM)] * 17,
        out_specs=pl.BlockSpec(memory_space=pltpu.MemorySpace.VMEM),
        out_shape=jax.ShapeDtypeStruct((B, 1), jnp.float32),
        scratch_shapes=[pltpu.VMEM((2, B, 128), jnp.float32),
                        pltpu.SemaphoreType.DMA((2,))],
    )(*args)


def kernel(batch_user, batch_label, table, W1, b1, alpha1, W2, b2, W3, b3,
           Wf1, bf1, af1, Wf2, bf2, af2, Wf3, bf3):
    bu = batch_user.astype(jnp.int32)
    bl = batch_label.astype(jnp.int32)
    idx_user = (bu >> 1).T.reshape(NW, NCHUNK, CHUNK)
    idx_label = (bl >> 1).reshape(-1)
    par = jnp.concatenate([bu & 1, bl & 1], axis=1).astype(jnp.float32)

    tpad = _transpose_pad_tc(table.T, jnp.eye(EMBED, dtype=jnp.float32))
    ue_flat, le = _gather_sc(tpad, idx_user, idx_label)
    ue = ue_flat.reshape(F, B, 128)

    return _tc_forward(ue, le, par, W1, b1, alpha1, W2, b2, W3, b3,
                       Wf1, bf1, af1, Wf2, bf2, af2, Wf3, bf3)


# speedup vs baseline: 6.2487x; 1.0154x over previous
import functools

import jax
import jax.numpy as jnp
from jax import lax
from jax.experimental import pallas as pl
from jax.experimental.pallas import tpu as pltpu
from jax.experimental.pallas import tpu_sc as plsc

ITEM_NUM = 1000000
EMBED = 96
FEATURE_GROUPS = [20, 20, 10, 10, 2, 2, 2, 1, 1, 1]
F = sum(FEATURE_GROUPS)
G = len(FEATURE_GROUPS)
B = 1024

NC, NS = 2, 16
NW = NC * NS
ROWS_W = (B * F) // NW
CHUNK = 96
NCHUNK = ROWS_W // CHUNK
LROWS = B // NW

BN_S = 0.9999950000374997


TBL = 32768
NTB = -(-(ITEM_NUM + 1) // TBL)
TROWS = NTB * TBL


def _transpose_pad_tc(tableT, eye):
    def body(tT_ref, eye_ref, out_ref):
        x = tT_ref[...]
        xT = x.T
        y = xT.astype(jnp.bfloat16)
        pf = pltpu.bitcast(y, jnp.float32)
        out_ref[:, :EMBED] = pf
        out_ref[:, EMBED:] = jnp.zeros((TBL // 2, 128 - EMBED), jnp.float32)

    return pl.pallas_call(
        body,
        grid=(NTB,),
        in_specs=[pl.BlockSpec((EMBED, TBL), lambda t: (0, t)),
                  pl.BlockSpec((EMBED, EMBED), lambda t: (0, 0))],
        out_specs=pl.BlockSpec((TBL // 2, 128), lambda t: (t, 0)),
        out_shape=jax.ShapeDtypeStruct((TROWS // 2, 128), jnp.float32),
        compiler_params=pltpu.CompilerParams(
            dimension_semantics=("parallel",)),
    )(tableT, eye)


def _gather_sc(table, idx_user, idx_label):
    mesh = plsc.VectorSubcoreMesh(core_axis_name="c", subcore_axis_name="s")

    @functools.partial(
        pl.kernel,
        mesh=mesh,
        out_type=[
            jax.ShapeDtypeStruct((B * F, 128), jnp.float32),
            jax.ShapeDtypeStruct((B, 128), jnp.float32),
        ],
        scratch_types=[
            pltpu.VMEM((NCHUNK, CHUNK), jnp.int32),
            pltpu.VMEM((CHUNK, 128), jnp.float32),
            pltpu.VMEM((CHUNK, 128), jnp.float32),
            pltpu.VMEM((LROWS,), jnp.int32),
            pltpu.VMEM((LROWS, 128), jnp.float32),
            pltpu.SemaphoreType.DMA,
            pltpu.SemaphoreType.DMA,
        ],
    )
    def k(table_hbm, idxu_hbm, idxl_hbm, out_u, out_l,
          idx_v, buf0, buf1, idxl_v, lbuf, gsem, wsem):
        wid = lax.axis_index("s") * NC + lax.axis_index("c")
        base = wid * ROWS_W

        pltpu.sync_copy(idxl_hbm.at[pl.ds(wid * LROWS, LROWS)], idxl_v)
        pltpu.async_copy(table_hbm.at[idxl_v], lbuf, gsem).wait()
        pltpu.sync_copy(lbuf, out_l.at[pl.ds(wid * LROWS, LROWS)])

        pltpu.sync_copy(idxu_hbm.at[wid], idx_v)
        bufs = (buf0, buf1)
        pending = [None, None]
        for c in range(NCHUNK):
            b = bufs[c % 2]
            if pending[c % 2] is not None:
                pending[c % 2].wait()
            pltpu.async_copy(table_hbm.at[idx_v.at[c]], b, gsem).wait()
            wb = pltpu.async_copy(
                b, out_u.at[pl.ds(base + c * CHUNK, CHUNK)], wsem)
            pending[c % 2] = wb
        pending[0].wait()
        pending[1].wait()

    return k(table, idx_user, idx_label)


def _dice(x, alpha):
    xp = 1.0 / (1.0 + jnp.exp(-x))
    return alpha * (1.0 - xp) * x + xp * x


def _tc_body(ue_hbm, le_ref, par_ref, W1_ref, b1_ref, alpha1_ref,
             W2_ref, b2_ref, W3_ref, b3_ref,
             Wf1_ref, bf1_ref, af1_ref, Wf2_ref, bf2_ref, af2_ref,
             Wf3_ref, bf3_ref, out_ref, ubuf, sems):
    def ucopy(f):
        return pltpu.make_async_copy(ue_hbm.at[f], ubuf.at[f % 2],
                                     sems.at[f % 2])

    def unpack(raw, par):
        wu = jax.lax.bitcast_convert_type(raw, jnp.uint32)
        lo = jax.lax.bitcast_convert_type(wu << 16, jnp.float32)
        hi = jax.lax.bitcast_convert_type(wu & jnp.uint32(0xFFFF0000),
                                          jnp.float32)
        return jnp.where(par > 0.5, hi, lo)

    def dot(a, b):
        return jnp.dot(a, b, preferred_element_type=jnp.float32)

    le = unpack(le_ref[...][:, :EMBED], par_ref[:, F:F + 1])
    W1 = W1_ref[...]
    Wq = W1[0:EMBED] + W1[2 * EMBED:3 * EMBED]
    Wum = jnp.concatenate([W1[EMBED:2 * EMBED] - W1[2 * EMBED:3 * EMBED],
                           W1[3 * EMBED:4 * EMBED]], axis=0)
    aq = dot(le, Wq) + b1_ref[...]
    alpha1 = alpha1_ref[...]
    W23 = dot(W2_ref[...], W3_ref[...])
    c23 = dot(b2_ref[...], W3_ref[...]) + b3_ref[...]

    ucopy(0).start()
    parts = [le]
    f = 0
    for n in FEATURE_GROUPS:
        psum = None
        for _ in range(n):
            if f + 1 < F:
                ucopy(f + 1).start()
            ucopy(f).wait()
            u = unpack(ubuf[f % 2][:, :EMBED], par_ref[:, f:f + 1])
            h1 = _dice(aq + dot(jnp.concatenate([u, le * u], axis=1), Wum),
                       alpha1)
            att = dot(h1, W23) + c23
            pre = u * att
            psum = pre if psum is None else psum + pre
            f += 1
        parts.append(psum)
    x = jnp.concatenate(parts[1:] + parts[:1], axis=1)
    h = (dot(x, Wf1_ref[...]) + bf1_ref[...]) * BN_S
    h = _dice(h, af1_ref[...])
    h = (dot(h, Wf2_ref[...]) + bf2_ref[...]) * BN_S
    h = _dice(h, af2_ref[...])
    out_ref[...] = dot(h, Wf3_ref[...]) + bf3_ref[...]


def _tc_forward(ue, le, par, W1, b1, alpha1, W2, b2, W3, b3,
                Wf1, bf1, af1, Wf2, bf2, af2, Wf3, bf3):
    args = (ue, le, par, W1, b1.reshape(1, -1), alpha1.reshape(1, -1),
            W2, b2.reshape(1, -1), W3, b3.reshape(1, -1),
            Wf1, bf1.reshape(1, -1), af1.reshape(1, -1),
            Wf2, bf2.reshape(1, -1), af2.reshape(1, -1),
            Wf3, bf3.reshape(1, -1))
    return pl.pallas_call(
        _tc_body,
        in_specs=[pl.BlockSpec(memory_space=pl.ANY)]
                 + [pl.BlockSpec(memory_space=pltpu.MemorySpace.VMEM)] * 17,
        out_specs=pl.BlockSpec(memory_space=pltpu.MemorySpace.VMEM),
        out_shape=jax.ShapeDtypeStruct((B, 1), jnp.float32),
        scratch_shapes=[pltpu.VMEM((2, B, 128), jnp.float32),
                        pltpu.SemaphoreType.DMA((2,))],
    )(*args)


def kernel(batch_user, batch_label, table, W1, b1, alpha1, W2, b2, W3, b3,
           Wf1, bf1, af1, Wf2, bf2, af2, Wf3, bf3):
    bu = batch_user.astype(jnp.int32)
    bl = batch_label.astype(jnp.int32)
    idx_user = (bu >> 1).T.reshape(NW, NCHUNK, CHUNK)
    idx_label = (bl >> 1).reshape(-1)
    par = jnp.concatenate([bu & 1, bl & 1], axis=1).astype(jnp.float32)

    tpad = _transpose_pad_tc(table.T, jnp.eye(EMBED, dtype=jnp.float32))
    ue_flat, le = _gather_sc(tpad, idx_user, idx_label)
    ue = ue_flat.reshape(F, B, 128)

    return _tc_forward(ue, le, par, W1, b1, alpha1, W2, b2, W3, b3,
                       Wf1, bf1, af1, Wf2, bf2, af2, Wf3, bf3)


# speedup vs baseline: 6.2822x; 1.0054x over previous
import functools

import jax
import jax.numpy as jnp
from jax import lax
from jax.experimental import pallas as pl
from jax.experimental.pallas import tpu as pltpu
from jax.experimental.pallas import tpu_sc as plsc

ITEM_NUM = 1000000
EMBED = 96
FEATURE_GROUPS = [20, 20, 10, 10, 2, 2, 2, 1, 1, 1]
F = sum(FEATURE_GROUPS)
G = len(FEATURE_GROUPS)
B = 1024

NC, NS = 2, 16
NW = NC * NS
ROWS_W = (B * F) // NW
CHUNK = 96
NCHUNK = ROWS_W // CHUNK
LROWS = B // NW

BN_S = 0.9999950000374997


TBL = 32768
NTB = -(-(ITEM_NUM + 1) // TBL)
TROWS = NTB * TBL


def _transpose_pad_tc(tableT):
    def body(tT_ref, out_ref):
        x = tT_ref[...]
        xT = x.T
        y = xT.astype(jnp.bfloat16)
        pf = pltpu.bitcast(y, jnp.float32)
        out_ref[:, :EMBED] = pf
        out_ref[:, EMBED:] = jnp.zeros((TBL // 2, 128 - EMBED), jnp.float32)

    return pl.pallas_call(
        body,
        grid=(NTB,),
        in_specs=[pl.BlockSpec((EMBED, TBL), lambda t: (0, t))],
        out_specs=pl.BlockSpec((TBL // 2, 128), lambda t: (t, 0)),
        out_shape=jax.ShapeDtypeStruct((TROWS // 2, 128), jnp.float32),
        compiler_params=pltpu.CompilerParams(
            dimension_semantics=("parallel",)),
    )(tableT)


def _gather_sc(table, idx_user, idx_label):
    mesh = plsc.VectorSubcoreMesh(core_axis_name="c", subcore_axis_name="s")

    @functools.partial(
        pl.kernel,
        mesh=mesh,
        out_type=[
            jax.ShapeDtypeStruct((B * F, 128), jnp.float32),
            jax.ShapeDtypeStruct((B, 128), jnp.float32),
        ],
        scratch_types=[
            pltpu.VMEM((NCHUNK, CHUNK), jnp.int32),
            pltpu.VMEM((CHUNK, 128), jnp.float32),
            pltpu.VMEM((CHUNK, 128), jnp.float32),
            pltpu.VMEM((LROWS,), jnp.int32),
            pltpu.VMEM((LROWS, 128), jnp.float32),
            pltpu.SemaphoreType.DMA,
            pltpu.SemaphoreType.DMA,
        ],
    )
    def k(table_hbm, idxu_hbm, idxl_hbm, out_u, out_l,
          idx_v, buf0, buf1, idxl_v, lbuf, gsem, wsem):
        wid = lax.axis_index("s") * NC + lax.axis_index("c")
        base = wid * ROWS_W

        pltpu.sync_copy(idxl_hbm.at[pl.ds(wid * LROWS, LROWS)], idxl_v)
        pltpu.async_copy(table_hbm.at[idxl_v], lbuf, gsem).wait()
        pltpu.sync_copy(lbuf, out_l.at[pl.ds(wid * LROWS, LROWS)])

        pltpu.sync_copy(idxu_hbm.at[wid], idx_v)
        bufs = (buf0, buf1)
        pending = [None, None]
        for c in range(NCHUNK):
            b = bufs[c % 2]
            if pending[c % 2] is not None:
                pending[c % 2].wait()
            pltpu.async_copy(table_hbm.at[idx_v.at[c]], b, gsem).wait()
            wb = pltpu.async_copy(
                b, out_u.at[pl.ds(base + c * CHUNK, CHUNK)], wsem)
            pending[c % 2] = wb
        pending[0].wait()
        pending[1].wait()

    return k(table, idx_user, idx_label)


def _dice(x, alpha):
    xp = 1.0 / (1.0 + jnp.exp(-x))
    return alpha * (1.0 - xp) * x + xp * x


def _tc_body(ue_hbm, le_ref, par_ref, W1_ref, b1_ref, alpha1_ref,
             W2_ref, b2_ref, W3_ref, b3_ref,
             Wf1_ref, bf1_ref, af1_ref, Wf2_ref, bf2_ref, af2_ref,
             Wf3_ref, bf3_ref, out_ref, ubuf, sems):
    def ucopy(f):
        return pltpu.make_async_copy(ue_hbm.at[f], ubuf.at[f % 2],
                                     sems.at[f % 2])

    def unpack(raw, par):
        wu = jax.lax.bitcast_convert_type(raw, jnp.uint32)
        lo = jax.lax.bitcast_convert_type(wu << 16, jnp.float32)
        hi = jax.lax.bitcast_convert_type(wu & jnp.uint32(0xFFFF0000),
                                          jnp.float32)
        return jnp.where(par > 0.5, hi, lo)

    def dot(a, b):
        return jnp.dot(a, b, preferred_element_type=jnp.float32)

    le = unpack(le_ref[...][:, :EMBED], par_ref[:, F:F + 1])
    W1 = W1_ref[...]
    Wq = W1[0:EMBED] + W1[2 * EMBED:3 * EMBED]
    Wum = jnp.concatenate([W1[EMBED:2 * EMBED] - W1[2 * EMBED:3 * EMBED],
                           W1[3 * EMBED:4 * EMBED]], axis=0)
    aq = dot(le, Wq) + b1_ref[...]
    alpha1 = alpha1_ref[...]
    W23 = dot(W2_ref[...], W3_ref[...])
    c23 = dot(b2_ref[...], W3_ref[...]) + b3_ref[...]

    ucopy(0).start()
    parts = [le]
    f = 0
    for n in FEATURE_GROUPS:
        psum = None
        for _ in range(n):
            if f + 1 < F:
                ucopy(f + 1).start()
            ucopy(f).wait()
            u = unpack(ubuf[f % 2][:, :EMBED], par_ref[:, f:f + 1])
            h1 = _dice(aq + dot(jnp.concatenate([u, le * u], axis=1), Wum),
                       alpha1)
            att = dot(h1, W23) + c23
            pre = u * att
            psum = pre if psum is None else psum + pre
            f += 1
        parts.append(psum)
    x = jnp.concatenate(parts[1:] + parts[:1], axis=1)
    h = (dot(x, Wf1_ref[...]) + bf1_ref[...]) * BN_S
    h = _dice(h, af1_ref[...])
    h = (dot(h, Wf2_ref[...]) + bf2_ref[...]) * BN_S
    h = _dice(h, af2_ref[...])
    out_ref[...] = dot(h, Wf3_ref[...]) + bf3_ref[...]


def _tc_forward(ue, le, par, W1, b1, alpha1, W2, b2, W3, b3,
                Wf1, bf1, af1, Wf2, bf2, af2, Wf3, bf3):
    args = (ue, le, par, W1, b1.reshape(1, -1), alpha1.reshape(1, -1),
            W2, b2.reshape(1, -1), W3, b3.reshape(1, -1),
            Wf1, bf1.reshape(1, -1), af1.reshape(1, -1),
            Wf2, bf2.reshape(1, -1), af2.reshape(1, -1),
            Wf3, bf3.reshape(1, -1))
    return pl.pallas_call(
        _tc_body,
        in_specs=[pl.BlockSpec(memory_space=pl.ANY)]
                 + [pl.BlockSpec(memory_space=pltpu.MemorySpace.VMEM)] * 17,
        out_specs=pl.BlockSpec(memory_space=pltpu.MemorySpace.VMEM),
        out_shape=jax.ShapeDtypeStruct((B, 1), jnp.float32),
        scratch_shapes=[pltpu.VMEM((2, B, 128), jnp.float32),
                        pltpu.SemaphoreType.DMA((2,))],
    )(*args)


def kernel(batch_user, batch_label, table, W1, b1, alpha1, W2, b2, W3, b3,
           Wf1, bf1, af1, Wf2, bf2, af2, Wf3, bf3):
    bu = batch_user.astype(jnp.int32)
    bl = batch_label.astype(jnp.int32)
    idx_user = (bu >> 1).T.reshape(NW, NCHUNK, CHUNK)
    idx_label = (bl >> 1).reshape(-1)
    par = jnp.concatenate([bu & 1, bl & 1], axis=1).astype(jnp.float32)

    tpad = _transpose_pad_tc(table.T)
    ue_flat, le = _gather_sc(tpad, idx_user, idx_label)
    ue = ue_flat.reshape(F, B, 128)

    return _tc_forward(ue, le, par, W1, b1, alpha1, W2, b2, W3, b3,
                       Wf1, bf1, af1, Wf2, bf2, af2, Wf3, bf3)
